# baseline (speedup 1.0000x reference)
import functools

import jax
import jax.numpy as jnp
from jax import lax
from jax.experimental import pallas as pl
from jax.experimental.pallas import tpu as pltpu
from jax.experimental.pallas import tpu_sc as plsc

_D = 64
_SCALE = 8.0


@functools.cache
def _build(BATCH, T, V):
    info = plsc.get_sparse_core_info()
    NC, NS = info.num_cores, info.num_subcores
    NW = NC * NS
    BW = BATCH // NW
    assert BATCH % NW == 0 and BW <= 128 and T % 2 == 0

    mesh = plsc.VectorSubcoreMesh(core_axis_name="c", subcore_axis_name="s")

    @functools.partial(
        pl.kernel,
        mesh=mesh,
        out_type=jax.ShapeDtypeStruct((T, _D, BATCH), jnp.float32),
        scratch_types=[
            pltpu.VMEM((T, BW), jnp.int32),
            pltpu.VMEM((2, BW, _D), jnp.float32),
            pltpu.VMEM((2, 1, _D, BW), jnp.float32),
            pltpu.SemaphoreType.DMA((2,)),
            pltpu.SemaphoreType.DMA((2,)),
        ],
        compiler_params=pltpu.CompilerParams(
            use_tc_tiling_on_sc=False, needs_layout_passes=False
        ),
    )
    def body(xt_hbm, t_hbm, out_hbm, idx_v, inb, outb, sg, so):
        wid = lax.axis_index("s") * NC + lax.axis_index("c")
        b0 = wid * BW
        iota = lax.iota(jnp.int32, 16)

        def fire_gather(s, t):
            pltpu.async_copy(t_hbm.at[idx_v.at[t]], inb.at[s], sg.at[s])

        def drain_gather(s):
            pltpu.make_async_copy(
                t_hbm.at[pl.ds(0, BW)], inb.at[s], sg.at[s]
            ).wait()

        def fire_out(s, t):
            pltpu.async_copy(
                outb.at[s],
                out_hbm.at[pl.ds(t, 1), pl.ds(0, _D), pl.ds(b0, BW)],
                so.at[s],
            )

        def wait_out(s):
            pltpu.make_async_copy(
                outb.at[s],
                out_hbm.at[pl.ds(0, 1), pl.ds(0, _D), pl.ds(0, BW)],
                so.at[s],
            ).wait()

        def transpose_scale(s):
            dst = outb.at[s, 0]

            def titer(bi, carry):
                for u in range(2):
                    b = bi * 2 + u
                    b_idx = jnp.full((16,), 0, jnp.int32) + b
                    for d0 in range(0, _D, 16):
                        vals = inb[s, b, pl.ds(d0, 16)] * _SCALE
                        plsc.store_scatter(dst, [iota + d0, b_idx], vals)
                return carry

            lax.fori_loop(0, BW // 2, titer, 0)

        pltpu.sync_copy(xt_hbm.at[pl.ds(0, T), pl.ds(b0, BW)], idx_v)
        fire_gather(0, 0)
        fire_gather(1, 1)

        for t in range(2):
            drain_gather(t)
            transpose_scale(t)
            fire_out(t, t)
            fire_gather(t, t + 2)

        def step(g, carry):
            for k in range(2):
                t = g * 2 + k
                drain_gather(k)
                wait_out(k)
                transpose_scale(k)
                fire_out(k, t)
                fire_gather(k, t + 2)
            return carry

        lax.fori_loop(1, T // 2 - 1, step, 0)

        for t in range(T - 2, T):
            k = t % 2
            drain_gather(k)
            wait_out(k)
            transpose_scale(k)
            fire_out(k, t)

        wait_out(0)
        wait_out(1)

    return body


def kernel(x, table):
    xt = jnp.swapaxes(x, 0, 1).astype(jnp.int32)
    out = _build(x.shape[0], x.shape[1], table.shape[0])(xt, table)
    return jnp.transpose(out, (2, 0, 1))


# speedup vs baseline: 1.5560x; 1.5560x over previous
import functools

import jax
import jax.numpy as jnp
from jax import lax
from jax.experimental import pallas as pl
from jax.experimental.pallas import tpu as pltpu
from jax.experimental.pallas import tpu_sc as plsc

_D = 64
_SCALE = 8.0
_SENT = 2
_NBUF = 3


@functools.cache
def _build(BATCH, T, V):
    info = plsc.get_sparse_core_info()
    NC, NS = info.num_cores, info.num_subcores
    NW = NC * NS
    sents_per_w = BATCH // NW
    n_chunks = sents_per_w // _SENT
    assert BATCH % NW == 0 and sents_per_w % _SENT == 0
    n_main = ((n_chunks - 4) // _NBUF) * _NBUF
    assert n_main >= _NBUF and n_chunks - n_main <= 6
    bursts = [(0, 128), (128, T - 128)] if T > 128 else [(0, T)]

    mesh = plsc.VectorSubcoreMesh(core_axis_name="c", subcore_axis_name="s")

    @functools.partial(
        pl.kernel,
        mesh=mesh,
        out_type=jax.ShapeDtypeStruct((BATCH, T, _D), jnp.float32),
        scratch_types=[
            pltpu.VMEM((sents_per_w, T), jnp.int32),
            pltpu.VMEM((_NBUF, _SENT, T, _D), jnp.float32),
            pltpu.SemaphoreType.DMA((_NBUF,)),
            pltpu.SemaphoreType.DMA((_NBUF,)),
        ],
        compiler_params=pltpu.CompilerParams(use_tc_tiling_on_sc=False),
    )
    def body(x_hbm, t_hbm, out_hbm, idx_v, bufs, sg, so):
        wid = lax.axis_index("s") * NC + lax.axis_index("c")
        sent_base = wid * sents_per_w

        def fire_gather(s, c):
            for q in range(_SENT):
                for (off, ln) in bursts:
                    pltpu.async_copy(
                        t_hbm.at[idx_v.at[c * _SENT + q, pl.ds(off, ln)]],
                        bufs.at[s, q, pl.ds(off, ln)],
                        sg.at[s],
                    )

        def drain(sem_row):
            pltpu.make_async_copy(
                out_hbm.at[pl.ds(0, _SENT)], bufs.at[0], sem_row
            ).wait()

        def scale(s):
            RU = 8

            def srow(q):
                def iter_(r, carry):
                    for u in range(RU):
                        t = r * RU + u
                        for j4 in range(_D // 16):
                            sl = pl.ds(16 * j4, 16)
                            bufs[s, q, t, sl] = bufs[s, q, t, sl] * _SCALE
                    return carry
                lax.fori_loop(0, T // RU, iter_, 0)
                for t in range(T - T % RU, T):
                    for j4 in range(_D // 16):
                        sl = pl.ds(16 * j4, 16)
                        bufs[s, q, t, sl] = bufs[s, q, t, sl] * _SCALE

            for q in range(_SENT):
                srow(q)

        def fire_out(s, c):
            sb = sent_base + c * _SENT
            pltpu.async_copy(bufs.at[s], out_hbm.at[pl.ds(sb, _SENT)], so.at[s])

        pltpu.sync_copy(x_hbm.at[pl.ds(sent_base, sents_per_w)], idx_v)
        fire_gather(0, 0)
        fire_gather(1, 1)

        drain(sg.at[0])
        scale(0)
        fire_gather(2, 2)
        fire_out(0, 0)

        drain(sg.at[1])
        scale(1)
        drain(so.at[0])
        fire_gather(0, 3)
        fire_out(1, 1)

        drain(sg.at[2])
        scale(2)
        drain(so.at[1])
        fire_gather(1, 4)
        fire_out(2, 2)

        def ring(g, carry):
            c0 = g * _NBUF
            for k in range(_NBUF):
                c = c0 + k
                drain(sg.at[k])
                scale(k)
                drain(so.at[(k + 2) % _NBUF])
                fire_gather((k + 2) % _NBUF, c + 2)
                fire_out(k, c)
            return carry

        lax.fori_loop(1, n_main // _NBUF, ring, 0)

        for c in range(n_main, n_chunks):
            s = c % _NBUF
            drain(sg.at[s])
            scale(s)
            drain(so.at[(s + 2) % _NBUF])
            if c + 2 < n_chunks:
                fire_gather((s + 2) % _NBUF, c + 2)
            fire_out(s, c)

        drain(so.at[(n_chunks - 1) % _NBUF])

    return body


def kernel(x, table):
    x32 = x.astype(jnp.int32)
    return _build(x.shape[0], x.shape[1], table.shape[0])(x32, table)
